# baseline (speedup 1.0000x reference)
import functools

import jax
import jax.numpy as jnp
from jax import lax
from jax.experimental import pallas as pl
from jax.experimental.pallas import tpu as pltpu
from jax.experimental.pallas import tpu_sc as plsc

VOCAB = 1000000
EMBED_DIM = 64
BATCH = 16384

_NUM_CORES = 2
_NUM_SUBCORES = 16
_NUM_WORKERS = _NUM_CORES * _NUM_SUBCORES
_B_PER_W = BATCH // _NUM_WORKERS
_LANES = 16
_N_GROUPS = _B_PER_W // _LANES


def _make_sc_gather():
    mesh = plsc.VectorSubcoreMesh(core_axis_name="c", subcore_axis_name="s")

    @functools.partial(
        pl.kernel,
        mesh=mesh,
        out_type=jax.ShapeDtypeStruct((BATCH, EMBED_DIM), jnp.float32),
        scratch_types=[
            pltpu.VMEM((_B_PER_W,), jnp.int32),
            pltpu.VMEM((_B_PER_W, EMBED_DIM), jnp.float32),
            pltpu.SemaphoreType.DMA,
        ],
        compiler_params=pltpu.CompilerParams(needs_layout_passes=False),
    )
    def k(idx_hbm, table_hbm, out_hbm, idx_v, rows_v, sem):
        wid = lax.axis_index("s") * _NUM_CORES + lax.axis_index("c")
        wbase = wid * _B_PER_W
        pltpu.sync_copy(idx_hbm.at[pl.ds(wbase, _B_PER_W)], idx_v)

        def fire_group(g, carry):
            for l in range(_LANES):
                pltpu.async_copy(
                    table_hbm.at[g + l], rows_v.at[g * _LANES + l], sem
                )
            return carry

        lax.fori_loop(0, _N_GROUPS, fire_group, 0)

        def drain(j, carry):
            pltpu.make_async_copy(table_hbm.at[0], rows_v.at[0], sem).wait()
            return carry

        lax.fori_loop(0, _B_PER_W, drain, 0)
        pltpu.sync_copy(rows_v, out_hbm.at[pl.ds(wbase, _B_PER_W)])

    return k


_sc_gather = _make_sc_gather()


@jax.jit
def kernel(category, table):
    return _sc_gather(category.astype(jnp.int32), table)


# speedup vs baseline: 1.0784x; 1.0784x over previous
import functools

import jax
import jax.numpy as jnp
from jax import lax
from jax.experimental import pallas as pl
from jax.experimental.pallas import tpu as pltpu
from jax.experimental.pallas import tpu_sc as plsc

VOCAB = 1000000
EMBED_DIM = 64
BATCH = 16384

_NUM_CORES = 2
_NUM_SUBCORES = 16
_NUM_WORKERS = _NUM_CORES * _NUM_SUBCORES
_B_PER_W = BATCH // _NUM_WORKERS
_LANES = 16
_N_GROUPS = _B_PER_W // _LANES


def _make_sc_gather():
    mesh = plsc.VectorSubcoreMesh(core_axis_name="c", subcore_axis_name="s")

    @functools.partial(
        pl.kernel,
        mesh=mesh,
        out_type=jax.ShapeDtypeStruct((BATCH, EMBED_DIM), jnp.float32),
        scratch_types=[
            pltpu.VMEM((_B_PER_W,), jnp.int32),
            pltpu.VMEM((_B_PER_W, EMBED_DIM), jnp.float32),
            pltpu.SemaphoreType.DMA,
        ],
    )
    def k(idx_hbm, table_hbm, out_hbm, idx_v, rows_v, sem):
        wid = lax.axis_index("s") * _NUM_CORES + lax.axis_index("c")
        wbase = wid * _B_PER_W
        pltpu.sync_copy(idx_hbm.at[pl.ds(wbase, _B_PER_W)], idx_v)

        def fire_group(g, carry):
            iv = idx_v[pl.ds(g * _LANES, _LANES)]
            for l in range(_LANES):
                r = iv[l]
                pltpu.async_copy(
                    table_hbm.at[r], rows_v.at[g * _LANES + l], sem
                )
            return carry

        lax.fori_loop(0, _N_GROUPS, fire_group, 0)

        def drain(j, carry):
            pltpu.make_async_copy(table_hbm.at[0], rows_v.at[0], sem).wait()
            return carry

        lax.fori_loop(0, _B_PER_W, drain, 0)
        pltpu.sync_copy(rows_v, out_hbm.at[pl.ds(wbase, _B_PER_W)])

    return k


_sc_gather = _make_sc_gather()


@jax.jit
def kernel(category, table):
    return _sc_gather(category.astype(jnp.int32), table)
